# baseline (speedup 1.0000x reference)
import jax
import jax.numpy as jnp
from jax.experimental import pallas as pl
from jax.experimental.pallas import tpu as pltpu

HID = 32
MLPH = 64
EPS = 1e-5
NUM_GRAPHS = 128
OUT_CH = 2

ENC_TILE = 4096
EDGE_TILE = 2048
TN = 2048
KC = 64
POOL_TILE = 4096
VMEM_LIMIT = 40 * 1024 * 1024


def _ru(n, m):
    return ((n + m - 1) // m) * m


def _dot(a, b):
    return jnp.dot(a.astype(jnp.bfloat16), b.astype(jnp.bfloat16),
                   preferred_element_type=jnp.float32)


def _layer_norm(x, g, b):
    mu = jnp.mean(x, axis=-1, keepdims=True)
    var = jnp.mean(jnp.square(x - mu), axis=-1, keepdims=True)
    return (x - mu) * jax.lax.rsqrt(var + EPS) * g + b


def _dot16(a, b):
    return jnp.dot(a, b,
                   preferred_element_type=jnp.float32).astype(jnp.bfloat16)


def _tail(h1, wh, brows, w4, b4, r0=1, w0=0):
    b16 = jnp.bfloat16
    h = jnp.maximum(h1.astype(b16), 0.0)
    h = jnp.maximum(_dot16(h, wh[w0:w0 + MLPH])
                    + brows[r0:r0 + 1].astype(b16), 0.0)
    h = jnp.maximum(_dot16(h, wh[w0 + MLPH:w0 + 2 * MLPH])
                    + brows[r0 + 1:r0 + 2].astype(b16), 0.0)
    h = _layer_norm(h.astype(jnp.float32), brows[r0 + 2:r0 + 3],
                    brows[r0 + 3:r0 + 4])
    return _dot(h, w4) + b4


def _enc_kernel(x_ref, w1_ref, wh_ref, w4_ref, b_ref, b4_ref, o_ref):
    h1 = _dot(x_ref[...], w1_ref[...]) + b_ref[0:1]
    o_ref[...] = _tail(h1, wh_ref, b_ref, w4_ref[...],
                       b4_ref[...]).astype(o_ref.dtype)


def _edge_kernel(xr_ref, xc_ref, e_ref, wxr_ref, wo_ref, wh_ref,
                 w4e_ref, b_ref, b4e_ref, w4n_ref, b4n_ref, eo_ref, mo_ref):
    xr, xc, e = xr_ref[...], xc_ref[...], e_ref[...]
    t = _dot(xr, wxr_ref[...])
    h1e = (t[:, 0:MLPH] + _dot(xc, wo_ref[0:HID])
           + _dot(e, wo_ref[HID:2 * HID]) + b_ref[0:1])
    e_new = e + _tail(h1e, wh_ref, b_ref, w4e_ref[...], b4e_ref[...],
                      r0=1, w0=0)
    eo_ref[...] = e_new
    h1n = (t[:, MLPH:2 * MLPH] + _dot(e_new, wo_ref[2 * HID:3 * HID])
           + b_ref[5:6])
    mo_ref[...] = _tail(h1n, wh_ref, b_ref, w4n_ref[...], b4n_ref[...],
                        r0=6, w0=2 * MLPH).astype(jnp.bfloat16)


def _node_kernel(lo_ref, hi_ref, col_ref, msg_ref, x_ref,
                 w1a_ref, wh_ref, w4_ref, b_ref, b4_ref, o_ref, acc_ref):
    t = pl.program_id(0)
    base = t * TN
    acc_ref[...] = jnp.zeros_like(acc_ref)

    def body(j, carry):
        ids = col_ref[j]
        seg = jax.lax.broadcasted_iota(jnp.int32, (TN, KC), 0) + base
        onehot = jnp.where(seg == ids, 1.0, 0.0).astype(jnp.bfloat16)
        acc_ref[...] += jnp.dot(onehot, msg_ref[pl.ds(j * KC, KC), :],
                                preferred_element_type=jnp.float32)
        return carry

    jax.lax.fori_loop(lo_ref[t], hi_ref[t], body, 0)
    x = x_ref[...]
    h1 = _dot(x, w1a_ref[...]) + acc_ref[...] + b_ref[0:1]
    o_ref[...] = (x.astype(jnp.float32)
                  + _tail(h1, wh_ref, b_ref, w4_ref[...],
                          b4_ref[...])).astype(o_ref.dtype)


def _pool_kernel(b_ref, x_ref, o_ref, acc_ref):
    k = pl.program_id(1)

    @pl.when(k == 0)
    def _():
        acc_ref[...] = jnp.zeros_like(acc_ref)

    ids = b_ref[0]
    seg = jax.lax.broadcasted_iota(jnp.int32, (NUM_GRAPHS, POOL_TILE), 0)
    onehot = jnp.where(seg == ids, 1.0, 0.0)
    acc_ref[...] += _dot(onehot, x_ref[...])

    @pl.when(k == pl.num_programs(1) - 1)
    def _():
        o_ref[0] = acc_ref[...]


def _decode_kernel(p_ref, w1_ref, b1_ref, ln2_ref, w2_ref, b2_ref, o_ref):
    xg = p_ref[0] + p_ref[1]
    mu = jnp.mean(xg, axis=-1, keepdims=True)
    var = jnp.mean(jnp.square(xg - mu), axis=-1, keepdims=True)
    xn = (xg - mu) * jax.lax.rsqrt(var + EPS)
    h = jnp.dot(xn, w1_ref[...],
                preferred_element_type=jnp.float32) + b1_ref[...]
    cols = []
    for c in range(OUT_CH):
        hc = h[:, c * HID:(c + 1) * HID]
        hn = _layer_norm(hc, ln2_ref[c:c + 1],
                         ln2_ref[OUT_CH + c:OUT_CH + c + 1])
        cols.append(jnp.sum(hn * w2_ref[c:c + 1], axis=-1, keepdims=True)
                    + b2_ref[:, c:c + 1])
    o_ref[...] = jnp.concatenate(cols, axis=1)


def _cparams(*sems):
    return pltpu.CompilerParams(dimension_semantics=sems,
                                vmem_limit_bytes=VMEM_LIMIT)


def _const_specs(ws):
    return [pl.BlockSpec(w.shape, lambda *_, nd=w.ndim: (0,) * nd)
            for w in ws]


def _mlp_rows(x, ws, tile, n_out, out_dtype=jnp.float32):
    rows, fin = x.shape
    return pl.pallas_call(
        _enc_kernel,
        grid=(rows // tile,),
        in_specs=[pl.BlockSpec((tile, fin), lambda i: (i, 0))]
                 + _const_specs(ws),
        out_specs=pl.BlockSpec((tile, n_out), lambda i: (i, 0)),
        out_shape=jax.ShapeDtypeStruct((rows, n_out), out_dtype),
        compiler_params=_cparams("parallel"),
    )(x, *ws)


def _edge_apply(g, e, ws):
    rows, h = e.shape
    nb = rows // EDGE_TILE
    return pl.pallas_call(
        _edge_kernel,
        grid=(nb,),
        in_specs=[pl.BlockSpec((EDGE_TILE, h), lambda i: (i, 0)),
                  pl.BlockSpec((EDGE_TILE, h), lambda i, n=nb: (i + n, 0)),
                  pl.BlockSpec((EDGE_TILE, h), lambda i: (i, 0))]
                 + _const_specs(ws),
        out_specs=(pl.BlockSpec((EDGE_TILE, h), lambda i: (i, 0)),
                   pl.BlockSpec((EDGE_TILE, MLPH), lambda i: (i, 0))),
        out_shape=(jax.ShapeDtypeStruct((rows, h), jnp.float32),
                   jax.ShapeDtypeStruct((rows, MLPH), jnp.bfloat16)),
        compiler_params=_cparams("parallel"),
    )(g, g, e, *ws)


def _node_apply(lo, hi, col3, msg, x, ws):
    n_pad, h = x.shape
    return pl.pallas_call(
        _node_kernel,
        grid_spec=pltpu.PrefetchScalarGridSpec(
            num_scalar_prefetch=2,
            grid=(n_pad // TN,),
            in_specs=[pl.BlockSpec(col3.shape, lambda i, *_: (0, 0, 0)),
                      pl.BlockSpec(msg.shape, lambda i, *_: (0, 0)),
                      pl.BlockSpec((TN, h), lambda i, *_: (i, 0))]
                     + [pl.BlockSpec(w.shape, lambda i, *_, nd=w.ndim:
                                     (0,) * nd) for w in ws],
            out_specs=pl.BlockSpec((TN, h), lambda i, *_: (i, 0)),
            scratch_shapes=[pltpu.VMEM((TN, MLPH), jnp.float32)],
        ),
        out_shape=jax.ShapeDtypeStruct((n_pad, h), x.dtype),
        compiler_params=_cparams("parallel"),
    )(lo, hi, col3, msg, x, *ws)


def _pool_apply(batch3, x, ncores):
    n_pad, h = x.shape
    tsteps = n_pad // POOL_TILE // ncores
    return pl.pallas_call(
        _pool_kernel,
        grid=(ncores, tsteps),
        in_specs=[pl.BlockSpec((1, 1, POOL_TILE),
                               lambda c, k, t=tsteps: (c * t + k, 0, 0)),
                  pl.BlockSpec((POOL_TILE, h),
                               lambda c, k, t=tsteps: (c * t + k, 0))],
        out_specs=pl.BlockSpec((1, NUM_GRAPHS, h), lambda c, k: (c, 0, 0)),
        out_shape=jax.ShapeDtypeStruct((ncores, NUM_GRAPHS, h), jnp.float32),
        scratch_shapes=[pltpu.VMEM((NUM_GRAPHS, h), jnp.float32)],
        compiler_params=_cparams("parallel", "arbitrary"),
    )(batch3, x)


def _decode_apply(pooled, ws):
    return pl.pallas_call(
        _decode_kernel,
        grid=(1,),
        in_specs=_const_specs([pooled] + ws),
        out_specs=pl.BlockSpec((NUM_GRAPHS, OUT_CH), lambda i: (0, 0)),
        out_shape=jax.ShapeDtypeStruct((NUM_GRAPHS, OUT_CH), jnp.float32),
        compiler_params=_cparams("arbitrary"),
    )(pooled, *ws)


def _impl(x_nodes, edge_index, edge_attr, batch,
          ne_w1, ne_wh, ne_w4, ne_bias, ne_b4,
          ee_w1, ee_wh, ee_w4, ee_bias, ee_b4,
          c0_e_wxr, c0_e_wo, c0_e_wh, c0_e_w4, c0_e_bias, c0_e_b4,
          c0_n_w1, c0_n_wh, c0_n_w4, c0_n_bias, c0_n_b4,
          c1_e_wxr, c1_e_wo, c1_e_wh, c1_e_w4, c1_e_bias, c1_e_b4,
          c1_n_w1, c1_n_wh, c1_n_w4, c1_n_bias, c1_n_b4,
          c2_e_wxr, c2_e_wo, c2_e_wh, c2_e_w4, c2_e_bias, c2_e_b4,
          c2_n_w1, c2_n_wh, c2_n_w4, c2_n_bias, c2_n_b4,
          dec_w1, dec_b1, dec_ln2, dec_w2, dec_b2):
    n = x_nodes.shape[0]
    m = edge_attr.shape[0]
    n_pad = _ru(n, max(TN, ENC_TILE, POOL_TILE))
    m_pad = _ru(m, max(KC, EDGE_TILE))

    convs = []
    for (e_wxr, e_wo, e_wh, e_w4, e_bias, e_b4,
         n_w1, n_wh, n_w4, n_bias, n_b4) in [
            (c0_e_wxr, c0_e_wo, c0_e_wh, c0_e_w4, c0_e_bias, c0_e_b4,
             c0_n_w1, c0_n_wh, c0_n_w4, c0_n_bias, c0_n_b4),
            (c1_e_wxr, c1_e_wo, c1_e_wh, c1_e_w4, c1_e_bias, c1_e_b4,
             c1_n_w1, c1_n_wh, c1_n_w4, c1_n_bias, c1_n_b4),
            (c2_e_wxr, c2_e_wo, c2_e_wh, c2_e_w4, c2_e_bias, c2_e_b4,
             c2_n_w1, c2_n_wh, c2_n_w4, c2_n_bias, c2_n_b4)]:
        w1b = n_w1[HID:2 * HID]
        bf = lambda w: w.astype(jnp.bfloat16)
        convs.append(dict(
            edge=[bf(e_wxr), bf(e_wo), bf(e_wh), bf(e_w4[0:MLPH]), e_bias,
                  e_b4[0:1], bf(jnp.dot(e_w4[MLPH:2 * MLPH], w1b)),
                  jnp.dot(e_b4[1:2], w1b)],
            node=[bf(n_w1[0:HID]), bf(n_wh), bf(n_w4), n_bias, n_b4]))

    col = edge_index[1].astype(jnp.int32)
    row = edge_index[0].astype(jnp.int32)
    col_s, order = jax.lax.sort_key_val(col, jnp.arange(m, dtype=jnp.int32))
    row_s = jnp.take(row, order, axis=0, mode='clip')
    ea_s = jnp.take(edge_attr.astype(jnp.float32), order, axis=0, mode='clip')

    col_p = jnp.pad(col_s, (0, m_pad - m), constant_values=n_pad)
    row_p = jnp.pad(row_s, (0, m_pad - m))
    ea_p = jnp.pad(ea_s, (0, m_pad - m))
    xp = jnp.pad(x_nodes.astype(jnp.float32), ((0, n_pad - n), (0, 0)))
    batch_p = jnp.pad(batch.astype(jnp.int32), (0, n_pad - n),
                      constant_values=-1)
    col_g = jnp.minimum(col_p, n_pad - 1)

    n_tiles = n_pad // TN
    bounds = jnp.searchsorted(
        col_p,
        jnp.arange(n_tiles + 1, dtype=jnp.int32) * TN).astype(jnp.int32)
    start, end = bounds[:-1], bounds[1:]
    lo = start // KC
    hi = jnp.where(end > start, (end - 1) // KC + 1, lo)

    col3 = col_p.reshape(m_pad // KC, 1, KC)
    batch3 = batch_p.reshape(n_pad // POOL_TILE, 1, POOL_TILE)

    bf = lambda w: w.astype(jnp.bfloat16)

    x = _mlp_rows(xp, [bf(ne_w1), bf(ne_wh), bf(ne_w4), ne_bias, ne_b4],
                  ENC_TILE, HID, out_dtype=jnp.bfloat16)

    e_feat = xp[:, jnp.array([0, 3])]
    e_in = jnp.concatenate(
        [ea_p.reshape(-1, 1),
         jnp.take(e_feat, row_p, axis=0, mode='clip')
         - jnp.take(e_feat, col_g, axis=0, mode='clip')],
        axis=-1)
    e = _mlp_rows(e_in, [bf(ee_w1), bf(ee_wh), bf(ee_w4), ee_bias, ee_b4],
                  EDGE_TILE, HID)

    rc = jnp.concatenate([row_p, col_g])
    for conv in convs:
        g = jnp.take(x, rc, axis=0, mode='clip')
        e, msg = _edge_apply(g, e, conv['edge'])
        x = _node_apply(lo, hi, col3, msg, x, conv['node'])

    ncores = 2 if (n_pad // POOL_TILE) % 2 == 0 else 1
    pooled = _pool_apply(batch3, x, ncores)
    if ncores == 1:
        pooled = jnp.concatenate([pooled, jnp.zeros_like(pooled)], axis=0)
    return _decode_apply(pooled,
                         [dec_w1, dec_b1, dec_ln2, dec_w2, dec_b2])


kernel = jax.jit(_impl)


# speedup vs baseline: 1.0833x; 1.0833x over previous
import jax
import jax.numpy as jnp
from jax.experimental import pallas as pl
from jax.experimental.pallas import tpu as pltpu

HID = 32
MLPH = 64
EPS = 1e-5
NUM_GRAPHS = 128
OUT_CH = 2

ENC_TILE = 4096
EDGE_TILE = 2048
TN = 2048
KC = 128
POOL_TILE = 4096
VMEM_LIMIT = 40 * 1024 * 1024


def _ru(n, m):
    return ((n + m - 1) // m) * m


def _dot(a, b):
    return jnp.dot(a.astype(jnp.bfloat16), b.astype(jnp.bfloat16),
                   preferred_element_type=jnp.float32)


def _layer_norm(x, g, b):
    mu = jnp.mean(x, axis=-1, keepdims=True)
    var = jnp.mean(jnp.square(x - mu), axis=-1, keepdims=True)
    return (x - mu) * jax.lax.rsqrt(var + EPS) * g + b


def _dot16(a, b):
    return jnp.dot(a, b,
                   preferred_element_type=jnp.float32).astype(jnp.bfloat16)


def _tail(h1, wh, brows, w4, b4, r0=1, w0=0):
    b16 = jnp.bfloat16
    h = jnp.maximum(h1.astype(b16), 0.0)
    h = jnp.maximum(_dot16(h, wh[w0:w0 + MLPH])
                    + brows[r0:r0 + 1].astype(b16), 0.0)
    h = jnp.maximum(_dot16(h, wh[w0 + MLPH:w0 + 2 * MLPH])
                    + brows[r0 + 1:r0 + 2].astype(b16), 0.0)
    h = _layer_norm(h.astype(jnp.float32), brows[r0 + 2:r0 + 3],
                    brows[r0 + 3:r0 + 4])
    return _dot(h, w4) + b4


def _enc_kernel(x_ref, w1_ref, wh_ref, w4_ref, b_ref, b4_ref, o_ref):
    h1 = _dot(x_ref[...], w1_ref[...]) + b_ref[0:1]
    o_ref[...] = _tail(h1, wh_ref, b_ref, w4_ref[...],
                       b4_ref[...]).astype(o_ref.dtype)


def _edge_kernel(xr_ref, xc_ref, e_ref, wxr_ref, wo_ref, wh_ref,
                 w4e_ref, b_ref, b4e_ref, w4n_ref, b4n_ref, eo_ref, mo_ref):
    xr, xc, e = xr_ref[...], xc_ref[...], e_ref[...]
    t = _dot(xr, wxr_ref[...])
    h1e = (t[:, 0:MLPH] + _dot(xc, wo_ref[0:HID])
           + _dot(e, wo_ref[HID:2 * HID]) + b_ref[0:1])
    e_new = e + _tail(h1e, wh_ref, b_ref, w4e_ref[...], b4e_ref[...],
                      r0=1, w0=0)
    eo_ref[...] = e_new
    h1n = (t[:, MLPH:2 * MLPH] + _dot(e_new, wo_ref[2 * HID:3 * HID])
           + b_ref[5:6])
    mo_ref[...] = _tail(h1n, wh_ref, b_ref, w4n_ref[...], b4n_ref[...],
                        r0=6, w0=2 * MLPH).astype(jnp.bfloat16)


def _node_kernel(lo_ref, hi_ref, col_ref, msg_ref, x_ref,
                 w1a_ref, wh_ref, w4_ref, b_ref, b4_ref, o_ref, acc_ref):
    t = pl.program_id(0)
    base = t * TN
    acc_ref[...] = jnp.zeros_like(acc_ref)

    def body(j, carry):
        ids = col_ref[j]
        seg = jax.lax.broadcasted_iota(jnp.int32, (TN, KC), 0) + base
        onehot = jnp.where(seg == ids, 1.0, 0.0).astype(jnp.bfloat16)
        acc_ref[...] += jnp.dot(onehot, msg_ref[pl.ds(j * KC, KC), :],
                                preferred_element_type=jnp.float32)
        return carry

    jax.lax.fori_loop(lo_ref[t], hi_ref[t], body, 0)
    x = x_ref[...]
    h1 = _dot(x, w1a_ref[...]) + acc_ref[...] + b_ref[0:1]
    o_ref[...] = (x.astype(jnp.float32)
                  + _tail(h1, wh_ref, b_ref, w4_ref[...],
                          b4_ref[...])).astype(o_ref.dtype)


def _pool_kernel(b_ref, x_ref, o_ref, acc_ref):
    k = pl.program_id(1)

    @pl.when(k == 0)
    def _():
        acc_ref[...] = jnp.zeros_like(acc_ref)

    ids = b_ref[0]
    seg = jax.lax.broadcasted_iota(jnp.int32, (NUM_GRAPHS, POOL_TILE), 0)
    onehot = jnp.where(seg == ids, 1.0, 0.0)
    acc_ref[...] += _dot(onehot, x_ref[...])

    @pl.when(k == pl.num_programs(1) - 1)
    def _():
        o_ref[0] = acc_ref[...]


def _decode_kernel(p_ref, w1_ref, b1_ref, ln2_ref, w2_ref, b2_ref, o_ref):
    xg = p_ref[0] + p_ref[1]
    mu = jnp.mean(xg, axis=-1, keepdims=True)
    var = jnp.mean(jnp.square(xg - mu), axis=-1, keepdims=True)
    xn = (xg - mu) * jax.lax.rsqrt(var + EPS)
    h = jnp.dot(xn, w1_ref[...],
                preferred_element_type=jnp.float32) + b1_ref[...]
    cols = []
    for c in range(OUT_CH):
        hc = h[:, c * HID:(c + 1) * HID]
        hn = _layer_norm(hc, ln2_ref[c:c + 1],
                         ln2_ref[OUT_CH + c:OUT_CH + c + 1])
        cols.append(jnp.sum(hn * w2_ref[c:c + 1], axis=-1, keepdims=True)
                    + b2_ref[:, c:c + 1])
    o_ref[...] = jnp.concatenate(cols, axis=1)


def _cparams(*sems):
    return pltpu.CompilerParams(dimension_semantics=sems,
                                vmem_limit_bytes=VMEM_LIMIT)


def _const_specs(ws):
    return [pl.BlockSpec(w.shape, lambda *_, nd=w.ndim: (0,) * nd)
            for w in ws]


def _mlp_rows(x, ws, tile, n_out, out_dtype=jnp.float32):
    rows, fin = x.shape
    return pl.pallas_call(
        _enc_kernel,
        grid=(rows // tile,),
        in_specs=[pl.BlockSpec((tile, fin), lambda i: (i, 0))]
                 + _const_specs(ws),
        out_specs=pl.BlockSpec((tile, n_out), lambda i: (i, 0)),
        out_shape=jax.ShapeDtypeStruct((rows, n_out), out_dtype),
        compiler_params=_cparams("parallel"),
    )(x, *ws)


def _edge_apply(g, e, ws):
    rows, h = e.shape
    nb = rows // EDGE_TILE
    return pl.pallas_call(
        _edge_kernel,
        grid=(nb,),
        in_specs=[pl.BlockSpec((EDGE_TILE, h), lambda i: (i, 0)),
                  pl.BlockSpec((EDGE_TILE, h), lambda i, n=nb: (i + n, 0)),
                  pl.BlockSpec((EDGE_TILE, h), lambda i: (i, 0))]
                 + _const_specs(ws),
        out_specs=(pl.BlockSpec((EDGE_TILE, h), lambda i: (i, 0)),
                   pl.BlockSpec((EDGE_TILE, MLPH), lambda i: (i, 0))),
        out_shape=(jax.ShapeDtypeStruct((rows, h), jnp.float32),
                   jax.ShapeDtypeStruct((rows, MLPH), jnp.bfloat16)),
        compiler_params=_cparams("parallel"),
    )(g, g, e, *ws)


def _node_apply(lo, hi, col3, msg, x, ws):
    n_pad, h = x.shape
    return pl.pallas_call(
        _node_kernel,
        grid_spec=pltpu.PrefetchScalarGridSpec(
            num_scalar_prefetch=2,
            grid=(n_pad // TN,),
            in_specs=[pl.BlockSpec(col3.shape, lambda i, *_: (0, 0, 0)),
                      pl.BlockSpec(msg.shape, lambda i, *_: (0, 0)),
                      pl.BlockSpec((TN, h), lambda i, *_: (i, 0))]
                     + [pl.BlockSpec(w.shape, lambda i, *_, nd=w.ndim:
                                     (0,) * nd) for w in ws],
            out_specs=pl.BlockSpec((TN, h), lambda i, *_: (i, 0)),
            scratch_shapes=[pltpu.VMEM((TN, MLPH), jnp.float32)],
        ),
        out_shape=jax.ShapeDtypeStruct((n_pad, h), x.dtype),
        compiler_params=_cparams("parallel"),
    )(lo, hi, col3, msg, x, *ws)


def _pool_apply(batch3, x, ncores):
    n_pad, h = x.shape
    tsteps = n_pad // POOL_TILE // ncores
    return pl.pallas_call(
        _pool_kernel,
        grid=(ncores, tsteps),
        in_specs=[pl.BlockSpec((1, 1, POOL_TILE),
                               lambda c, k, t=tsteps: (c * t + k, 0, 0)),
                  pl.BlockSpec((POOL_TILE, h),
                               lambda c, k, t=tsteps: (c * t + k, 0))],
        out_specs=pl.BlockSpec((1, NUM_GRAPHS, h), lambda c, k: (c, 0, 0)),
        out_shape=jax.ShapeDtypeStruct((ncores, NUM_GRAPHS, h), jnp.float32),
        scratch_shapes=[pltpu.VMEM((NUM_GRAPHS, h), jnp.float32)],
        compiler_params=_cparams("parallel", "arbitrary"),
    )(batch3, x)


def _decode_apply(pooled, ws):
    return pl.pallas_call(
        _decode_kernel,
        grid=(1,),
        in_specs=_const_specs([pooled] + ws),
        out_specs=pl.BlockSpec((NUM_GRAPHS, OUT_CH), lambda i: (0, 0)),
        out_shape=jax.ShapeDtypeStruct((NUM_GRAPHS, OUT_CH), jnp.float32),
        compiler_params=_cparams("arbitrary"),
    )(pooled, *ws)


def _impl(x_nodes, edge_index, edge_attr, batch,
          ne_w1, ne_wh, ne_w4, ne_bias, ne_b4,
          ee_w1, ee_wh, ee_w4, ee_bias, ee_b4,
          c0_e_wxr, c0_e_wo, c0_e_wh, c0_e_w4, c0_e_bias, c0_e_b4,
          c0_n_w1, c0_n_wh, c0_n_w4, c0_n_bias, c0_n_b4,
          c1_e_wxr, c1_e_wo, c1_e_wh, c1_e_w4, c1_e_bias, c1_e_b4,
          c1_n_w1, c1_n_wh, c1_n_w4, c1_n_bias, c1_n_b4,
          c2_e_wxr, c2_e_wo, c2_e_wh, c2_e_w4, c2_e_bias, c2_e_b4,
          c2_n_w1, c2_n_wh, c2_n_w4, c2_n_bias, c2_n_b4,
          dec_w1, dec_b1, dec_ln2, dec_w2, dec_b2):
    n = x_nodes.shape[0]
    m = edge_attr.shape[0]
    n_pad = _ru(n, max(TN, ENC_TILE, POOL_TILE))
    m_pad = _ru(m, max(KC, EDGE_TILE))

    convs = []
    for (e_wxr, e_wo, e_wh, e_w4, e_bias, e_b4,
         n_w1, n_wh, n_w4, n_bias, n_b4) in [
            (c0_e_wxr, c0_e_wo, c0_e_wh, c0_e_w4, c0_e_bias, c0_e_b4,
             c0_n_w1, c0_n_wh, c0_n_w4, c0_n_bias, c0_n_b4),
            (c1_e_wxr, c1_e_wo, c1_e_wh, c1_e_w4, c1_e_bias, c1_e_b4,
             c1_n_w1, c1_n_wh, c1_n_w4, c1_n_bias, c1_n_b4),
            (c2_e_wxr, c2_e_wo, c2_e_wh, c2_e_w4, c2_e_bias, c2_e_b4,
             c2_n_w1, c2_n_wh, c2_n_w4, c2_n_bias, c2_n_b4)]:
        w1b = n_w1[HID:2 * HID]
        bf = lambda w: w.astype(jnp.bfloat16)
        convs.append(dict(
            edge=[bf(e_wxr), bf(e_wo), bf(e_wh), bf(e_w4[0:MLPH]), e_bias,
                  e_b4[0:1], bf(jnp.dot(e_w4[MLPH:2 * MLPH], w1b)),
                  jnp.dot(e_b4[1:2], w1b)],
            node=[bf(n_w1[0:HID]), bf(n_wh), bf(n_w4), n_bias, n_b4]))

    col = edge_index[1].astype(jnp.int32)
    row = edge_index[0].astype(jnp.int32)
    col_s, order = jax.lax.sort_key_val(col, jnp.arange(m, dtype=jnp.int32))
    row_s = jnp.take(row, order, axis=0, mode='clip')
    ea_s = jnp.take(edge_attr.astype(jnp.float32), order, axis=0, mode='clip')

    col_p = jnp.pad(col_s, (0, m_pad - m), constant_values=n_pad)
    row_p = jnp.pad(row_s, (0, m_pad - m))
    ea_p = jnp.pad(ea_s, (0, m_pad - m))
    xp = jnp.pad(x_nodes.astype(jnp.float32), ((0, n_pad - n), (0, 0)))
    batch_p = jnp.pad(batch.astype(jnp.int32), (0, n_pad - n),
                      constant_values=-1)
    col_g = jnp.minimum(col_p, n_pad - 1)

    n_tiles = n_pad // TN
    bounds = jnp.searchsorted(
        col_p,
        jnp.arange(n_tiles + 1, dtype=jnp.int32) * TN).astype(jnp.int32)
    start, end = bounds[:-1], bounds[1:]
    lo = start // KC
    hi = jnp.where(end > start, (end - 1) // KC + 1, lo)

    col3 = col_p.reshape(m_pad // KC, 1, KC)
    batch3 = batch_p.reshape(n_pad // POOL_TILE, 1, POOL_TILE)

    bf = lambda w: w.astype(jnp.bfloat16)

    x = _mlp_rows(xp, [bf(ne_w1), bf(ne_wh), bf(ne_w4), ne_bias, ne_b4],
                  ENC_TILE, HID, out_dtype=jnp.bfloat16)

    e_feat = xp[:, jnp.array([0, 3])]
    e_in = jnp.concatenate(
        [ea_p.reshape(-1, 1),
         jnp.take(e_feat, row_p, axis=0, mode='clip')
         - jnp.take(e_feat, col_g, axis=0, mode='clip')],
        axis=-1)
    e = _mlp_rows(e_in, [bf(ee_w1), bf(ee_wh), bf(ee_w4), ee_bias, ee_b4],
                  EDGE_TILE, HID)

    rc = jnp.concatenate([row_p, col_g])
    for conv in convs:
        g = jnp.take(x, rc, axis=0, mode='clip')
        e, msg = _edge_apply(g, e, conv['edge'])
        x = _node_apply(lo, hi, col3, msg, x, conv['node'])

    ncores = 2 if (n_pad // POOL_TILE) % 2 == 0 else 1
    pooled = _pool_apply(batch3, x, ncores)
    if ncores == 1:
        pooled = jnp.concatenate([pooled, jnp.zeros_like(pooled)], axis=0)
    return _decode_apply(pooled,
                         [dec_w1, dec_b1, dec_ln2, dec_w2, dec_b2])


kernel = jax.jit(_impl)
